# baseline (speedup 1.0000x reference)
import functools

import jax
import jax.numpy as jnp
from jax import lax
from jax.experimental import pallas as pl
from jax.experimental.pallas import tpu as pltpu
from jax.experimental.pallas import tpu_sc as plsc

_B = 1024
_NSPLIT = 2
_BH = _B // _NSPLIT
_A = 100
_AP = 104
_ATTRS = 23
_EMB = 32
_FAN = _ATTRS * _EMB
_RPH = _BH * _AP
_HID = 512
_NC, _NS = 2, 16
_NW = _NC * _NS
_CB = 2
_NCHUNK = _BH // (_NW * _CB)
_SLOT = _AP * _ATTRS
_SLOTP = 2400

_mesh = functools.partial(plsc.VectorSubcoreMesh,
                          core_axis_name="c", subcore_axis_name="s")

_SC_PARAMS = pltpu.CompilerParams(
    use_tc_tiling_on_sc=False, needs_layout_passes=False
)


def _wid():
    return lax.axis_index("s") * _NC + lax.axis_index("c")


def _make_gather(h):
    @functools.partial(
        pl.kernel,
        mesh=_mesh(),
        compiler_params=_SC_PARAMS,
        out_type=(
            jax.ShapeDtypeStruct((_RPH, _FAN), jnp.float32),
            jax.ShapeDtypeStruct((_BH * 128,), jnp.int32),
        ),
        scratch_types=[
            pltpu.VMEM((_CB * _SLOTP,), jnp.int32),
            pltpu.VMEM((_CB * _SLOTP,), jnp.int32),
            pltpu.VMEM((_CB * _SLOTP,), jnp.int32),
            pltpu.VMEM((_CB * _SLOTP,), jnp.int32),
            pltpu.VMEM((_CB, _A, _ATTRS), jnp.int32),
            pltpu.VMEM((_CB * _SLOTP,), jnp.int32),
            pltpu.VMEM((_SLOT, _EMB), jnp.float32),
            pltpu.SemaphoreType.DMA,
            pltpu.SemaphoreType.DMA,
        ],
        name=f"sc_gather_{h}",
    )
    def _gather(ag_hbm, pq_hbm, pr_hbm, pj_hbm, off_hbm, table_hbm,
                out_hbm, ids_hbm,
                pq_v, pr_v, pj_v, off_v, ag_v, idx_v, rows_v, sem_g, sem_o):
        pltpu.sync_copy(pq_hbm, pq_v)
        pltpu.sync_copy(pr_hbm, pr_v)
        pltpu.sync_copy(pj_hbm, pj_v)
        pltpu.sync_copy(off_hbm, off_v)
        w = _wid()

        def chunk_body(ci, carry):
            gc = w * _NCHUNK + ci
            pltpu.sync_copy(
                ag_hbm.at[pl.ds((h * _BH // _CB + gc) * _CB, _CB)], ag_v
            )

            def reorder(s, c2):
                s16 = s * 16
                sl = pl.ds(s16, 16)
                vals = plsc.load_gather(
                    ag_v, [pq_v[sl], pr_v[sl], pj_v[sl]]
                )
                vals = jnp.minimum(jnp.maximum(vals, 0), 255)
                idx_v[sl] = vals + off_v[sl]
                return c2

            lax.fori_loop(0, (_CB * _SLOTP) // 16, reorder, 0)

            def batch_body(q, c3):
                gb = gc * _CB + q
                pltpu.sync_copy(
                    idx_v.at[pl.ds(q * _SLOTP, _AP)],
                    ids_hbm.at[pl.ds(gb * 128, _AP)],
                )
                gathers = [
                    pltpu.async_copy(
                        table_hbm.at[
                            idx_v.at[pl.ds(q * _SLOTP + j * _AP, _AP)]
                        ],
                        rows_v.at[pl.ds(j * _AP, _AP)],
                        sem_g,
                    )
                    for j in range(_ATTRS)
                ]
                for c in gathers:
                    c.wait()
                outs = [
                    pltpu.async_copy(
                        rows_v.at[pl.ds(j * _AP, _AP)],
                        out_hbm.at[
                            pl.ds(gb * _AP, _AP), pl.ds(j * _EMB, _EMB)
                        ],
                        sem_o,
                    )
                    for j in range(_ATTRS)
                ]
                for c in outs:
                    c.wait()
                return c3

            lax.fori_loop(0, _CB, batch_body, 0)
            return carry

        lax.fori_loop(0, _NCHUNK, chunk_body, 0)

    return _gather


_sc_gather = [_make_gather(h) for h in range(_NSPLIT)]


_BWH = _BH // _NW


def _make_my_gather(h):
    @functools.partial(
        pl.kernel,
        mesh=_mesh(),
        compiler_params=_SC_PARAMS,
        out_type=jax.ShapeDtypeStruct((_BH, _FAN), jnp.float32),
        scratch_types=[
            pltpu.VMEM((_BWH, 8), jnp.int32),
            pltpu.VMEM((_BWH,), jnp.int32),
            pltpu.VMEM((_BWH, _FAN), jnp.float32),
            pltpu.SemaphoreType.DMA,
        ],
        name=f"sc_my_gather_{h}",
    )
    def _my_gather(g_hbm, emb_hbm, out_hbm, g8_v, g_v, rows_v, sem):
        base = _wid() * _BWH
        pltpu.sync_copy(
            g_hbm.at[pl.ds(h * _BH + base, _BWH), pl.ds(0, 8)], g8_v
        )
        for t in range(_BWH // 16):
            pos = lax.iota(jnp.int32, 16) + t * 16
            zero = jnp.zeros((16,), jnp.int32)
            g_v[pl.ds(t * 16, 16)] = (
                plsc.load_gather(g8_v, [pos, zero]) - h * _RPH
            )
        pltpu.async_copy(emb_hbm.at[g_v], rows_v, sem).wait()
        pltpu.sync_copy(rows_v, out_hbm.at[pl.ds(base, _BWH)])

    return _my_gather


_sc_my_gather = [_make_my_gather(h) for h in range(_NSPLIT)]


def _rowsel_body(ids_ref, my_ref, g_ref):
    ids = ids_ref[...]
    my = my_ref[...][:, :1]
    lane = lax.broadcasted_iota(jnp.int32, ids.shape, 1)
    match = (ids == my) & (ids != 0) & (lane < _A)
    cand = jnp.where(match, lane, 16384)
    row = jnp.min(cand, axis=1, keepdims=True)
    row = jnp.where(row >= 16384, 0, row)
    bidx = lax.broadcasted_iota(jnp.int32, row.shape, 0)
    g_ref[...] = jnp.broadcast_to(bidx * _AP + row, ids.shape)


def _rowsel(ids2, my8):
    return pl.pallas_call(
        _rowsel_body,
        out_shape=jax.ShapeDtypeStruct((_B, 128), jnp.int32),
    )(ids2, my8)


_BB = 8
_XB = _BB * _AP
_DN = (((1,), (1,)), ((), ()))


def _mm_body(x_ref, w_ref, b_ref, o_ref):
    res = (
        lax.dot_general(
            x_ref[...], w_ref[...], _DN,
            preferred_element_type=jnp.float32,
        )
        + b_ref[...]
    )
    for k in range(_BB):
        o_ref[k] = res[k * _AP : k * _AP + _A]


def _mm(emb, w, bias):
    return pl.pallas_call(
        _mm_body,
        grid=(_BH // _BB,),
        in_specs=[
            pl.BlockSpec((_XB, _FAN), lambda i: (i, 0)),
            pl.BlockSpec((_HID, _FAN), lambda i: (0, 0)),
            pl.BlockSpec((1, _HID), lambda i: (0, 0)),
        ],
        out_specs=pl.BlockSpec((_BB, _A, _HID), lambda i: (i, 0, 0)),
        out_shape=jax.ShapeDtypeStruct((_BH, _A, _HID), jnp.float32),
        compiler_params=pltpu.CompilerParams(
            dimension_semantics=("arbitrary",)
        ),
    )(emb, w, bias)


def _myfc_body(x_ref, w_ref, b_ref, o_ref):
    o_ref[...] = jnp.maximum(
        lax.dot_general(
            x_ref[...], w_ref[...], _DN,
            preferred_element_type=jnp.float32,
        )
        + b_ref[...],
        0.0,
    )


def _myfc(x, w, bias):
    return pl.pallas_call(
        _myfc_body,
        out_shape=jax.ShapeDtypeStruct((_B, _HID), jnp.float32),
    )(x, w, bias)


def _make_maps():
    s = jnp.arange(_CB * _SLOTP, dtype=jnp.int32)
    q = s // _SLOTP
    t = s % _SLOTP
    j = t // _AP
    r = t % _AP
    valid = (j < _ATTRS) & (r < _A)
    pq = jnp.where(valid, q, 0)
    pr = jnp.where(valid, r, 0)
    pj = jnp.where(valid, j, 0)
    off = jnp.where(j < _ATTRS, j * 256, 0)
    return pq, pr, pj, off


def kernel(agents, my_id, emb_table, agent_w, agent_b, my_w, my_b):
    pq, pr, pj, off = _make_maps()
    halves = [
        _sc_gather[h](agents, pq, pr, pj, off, emb_table)
        for h in range(_NSPLIT)
    ]

    my8 = jnp.broadcast_to(my_id[:, None], (_B, 8))
    ids2 = jnp.concatenate(
        [ids.reshape(_BH, 128) for _, ids in halves], axis=0
    )
    g2 = _rowsel(ids2, my8)

    my_emb = jnp.concatenate(
        [_sc_my_gather[h](g2, halves[h][0]) for h in range(_NSPLIT)],
        axis=0,
    )

    agent_out = jnp.concatenate(
        [_mm(halves[h][0], agent_w, agent_b[None, :])
         for h in range(_NSPLIT)],
        axis=0,
    )
    my_out = _myfc(my_emb, my_w, my_b[None, :])
    return agent_out, my_out


# speedup vs baseline: 1.1373x; 1.1373x over previous
import functools

import jax
import jax.numpy as jnp
from jax import lax
from jax.experimental import pallas as pl
from jax.experimental.pallas import tpu as pltpu
from jax.experimental.pallas import tpu_sc as plsc

_B = 1024
_NSPLIT = 1
_BH = _B // _NSPLIT
_A = 100
_AP = 104
_ATTRS = 23
_EMB = 32
_FAN = _ATTRS * _EMB
_RPH = _BH * _AP
_HID = 512
_NC, _NS = 2, 16
_NW = _NC * _NS
_CB = 2
_NCHUNK = _BH // (_NW * _CB)
_SLOT = _AP * _ATTRS
_SLOTP = 2400

_mesh = functools.partial(plsc.VectorSubcoreMesh,
                          core_axis_name="c", subcore_axis_name="s")

_SC_PARAMS = pltpu.CompilerParams(
    use_tc_tiling_on_sc=False, needs_layout_passes=False
)


def _wid():
    return lax.axis_index("s") * _NC + lax.axis_index("c")


def _make_gather(h):
    @functools.partial(
        pl.kernel,
        mesh=_mesh(),
        compiler_params=_SC_PARAMS,
        out_type=(
            jax.ShapeDtypeStruct((_RPH, _FAN), jnp.float32),
            jax.ShapeDtypeStruct((_BH * 128,), jnp.int32),
        ),
        scratch_types=[
            pltpu.VMEM((_CB * _SLOTP,), jnp.int32),
            pltpu.VMEM((_CB * _SLOTP,), jnp.int32),
            pltpu.VMEM((_CB * _SLOTP,), jnp.int32),
            pltpu.VMEM((_CB * _SLOTP,), jnp.int32),
            pltpu.VMEM((_CB, _A, _ATTRS), jnp.int32),
            pltpu.VMEM((_CB * _SLOTP,), jnp.int32),
            pltpu.VMEM((_SLOT, _EMB), jnp.float32),
            pltpu.SemaphoreType.DMA,
            pltpu.SemaphoreType.DMA,
        ],
        name=f"sc_gather_{h}",
    )
    def _gather(ag_hbm, pq_hbm, pr_hbm, pj_hbm, off_hbm, table_hbm,
                out_hbm, ids_hbm,
                pq_v, pr_v, pj_v, off_v, ag_v, idx_v, rows_v, sem_g, sem_o):
        pltpu.sync_copy(pq_hbm, pq_v)
        pltpu.sync_copy(pr_hbm, pr_v)
        pltpu.sync_copy(pj_hbm, pj_v)
        pltpu.sync_copy(off_hbm, off_v)
        w = _wid()

        def chunk_body(ci, carry):
            gc = w * _NCHUNK + ci
            pltpu.sync_copy(
                ag_hbm.at[pl.ds((h * _BH // _CB + gc) * _CB, _CB)], ag_v
            )

            def reorder(s, c2):
                s16 = s * 16
                sl = pl.ds(s16, 16)
                vals = plsc.load_gather(
                    ag_v, [pq_v[sl], pr_v[sl], pj_v[sl]]
                )
                vals = jnp.minimum(jnp.maximum(vals, 0), 255)
                idx_v[sl] = vals + off_v[sl]
                return c2

            lax.fori_loop(0, (_CB * _SLOTP) // 16, reorder, 0)

            def batch_body(q, c3):
                gb = gc * _CB + q
                pltpu.sync_copy(
                    idx_v.at[pl.ds(q * _SLOTP, _AP)],
                    ids_hbm.at[pl.ds(gb * 128, _AP)],
                )
                gathers = [
                    pltpu.async_copy(
                        table_hbm.at[
                            idx_v.at[pl.ds(q * _SLOTP + j * _AP, _AP)]
                        ],
                        rows_v.at[pl.ds(j * _AP, _AP)],
                        sem_g,
                    )
                    for j in range(_ATTRS)
                ]
                for c in gathers:
                    c.wait()
                outs = [
                    pltpu.async_copy(
                        rows_v.at[pl.ds(j * _AP, _AP)],
                        out_hbm.at[
                            pl.ds(gb * _AP, _AP), pl.ds(j * _EMB, _EMB)
                        ],
                        sem_o,
                    )
                    for j in range(_ATTRS)
                ]
                for c in outs:
                    c.wait()
                return c3

            lax.fori_loop(0, _CB, batch_body, 0)
            return carry

        lax.fori_loop(0, _NCHUNK, chunk_body, 0)

    return _gather


_sc_gather = [_make_gather(h) for h in range(_NSPLIT)]


_BWH = _BH // _NW


def _make_my_gather(h):
    @functools.partial(
        pl.kernel,
        mesh=_mesh(),
        compiler_params=_SC_PARAMS,
        out_type=jax.ShapeDtypeStruct((_BH, _FAN), jnp.float32),
        scratch_types=[
            pltpu.VMEM((_BWH, 8), jnp.int32),
            pltpu.VMEM((_BWH,), jnp.int32),
            pltpu.VMEM((_BWH, _FAN), jnp.float32),
            pltpu.SemaphoreType.DMA,
        ],
        name=f"sc_my_gather_{h}",
    )
    def _my_gather(g_hbm, emb_hbm, out_hbm, g8_v, g_v, rows_v, sem):
        base = _wid() * _BWH
        pltpu.sync_copy(
            g_hbm.at[pl.ds(h * _BH + base, _BWH), pl.ds(0, 8)], g8_v
        )
        for t in range(_BWH // 16):
            pos = lax.iota(jnp.int32, 16) + t * 16
            zero = jnp.zeros((16,), jnp.int32)
            g_v[pl.ds(t * 16, 16)] = (
                plsc.load_gather(g8_v, [pos, zero]) - h * _RPH
            )
        pltpu.async_copy(emb_hbm.at[g_v], rows_v, sem).wait()
        pltpu.sync_copy(rows_v, out_hbm.at[pl.ds(base, _BWH)])

    return _my_gather


_sc_my_gather = [_make_my_gather(h) for h in range(_NSPLIT)]


def _rowsel_body(ids_ref, my_ref, g_ref):
    ids = ids_ref[...]
    my = my_ref[...][:, :1]
    lane = lax.broadcasted_iota(jnp.int32, ids.shape, 1)
    match = (ids == my) & (ids != 0) & (lane < _A)
    cand = jnp.where(match, lane, 16384)
    row = jnp.min(cand, axis=1, keepdims=True)
    row = jnp.where(row >= 16384, 0, row)
    bidx = lax.broadcasted_iota(jnp.int32, row.shape, 0)
    g_ref[...] = jnp.broadcast_to(bidx * _AP + row, ids.shape)


def _rowsel(ids2, my8):
    return pl.pallas_call(
        _rowsel_body,
        out_shape=jax.ShapeDtypeStruct((_B, 128), jnp.int32),
    )(ids2, my8)


_BB = 8
_XB = _BB * _AP
_DN = (((1,), (1,)), ((), ()))


def _mm_body(x_ref, w_ref, b_ref, o_ref):
    res = (
        lax.dot_general(
            x_ref[...], w_ref[...], _DN,
            preferred_element_type=jnp.float32,
        )
        + b_ref[...]
    )
    for k in range(_BB):
        o_ref[k] = res[k * _AP : k * _AP + _A]


def _mm(emb, w, bias):
    return pl.pallas_call(
        _mm_body,
        grid=(_BH // _BB,),
        in_specs=[
            pl.BlockSpec((_XB, _FAN), lambda i: (i, 0)),
            pl.BlockSpec((_HID, _FAN), lambda i: (0, 0)),
            pl.BlockSpec((1, _HID), lambda i: (0, 0)),
        ],
        out_specs=pl.BlockSpec((_BB, _A, _HID), lambda i: (i, 0, 0)),
        out_shape=jax.ShapeDtypeStruct((_BH, _A, _HID), jnp.float32),
        compiler_params=pltpu.CompilerParams(
            dimension_semantics=("arbitrary",)
        ),
    )(emb, w, bias)


def _myfc_body(x_ref, w_ref, b_ref, o_ref):
    o_ref[...] = jnp.maximum(
        lax.dot_general(
            x_ref[...], w_ref[...], _DN,
            preferred_element_type=jnp.float32,
        )
        + b_ref[...],
        0.0,
    )


def _myfc(x, w, bias):
    return pl.pallas_call(
        _myfc_body,
        out_shape=jax.ShapeDtypeStruct((_B, _HID), jnp.float32),
    )(x, w, bias)


def _make_maps():
    s = jnp.arange(_CB * _SLOTP, dtype=jnp.int32)
    q = s // _SLOTP
    t = s % _SLOTP
    j = t // _AP
    r = t % _AP
    valid = (j < _ATTRS) & (r < _A)
    pq = jnp.where(valid, q, 0)
    pr = jnp.where(valid, r, 0)
    pj = jnp.where(valid, j, 0)
    off = jnp.where(j < _ATTRS, j * 256, 0)
    return pq, pr, pj, off


def kernel(agents, my_id, emb_table, agent_w, agent_b, my_w, my_b):
    pq, pr, pj, off = _make_maps()
    halves = [
        _sc_gather[h](agents, pq, pr, pj, off, emb_table)
        for h in range(_NSPLIT)
    ]

    my8 = jnp.broadcast_to(my_id[:, None], (_B, 8))
    ids2 = jnp.concatenate(
        [ids.reshape(_BH, 128) for _, ids in halves], axis=0
    )
    g2 = _rowsel(ids2, my8)

    my_emb = jnp.concatenate(
        [_sc_my_gather[h](g2, halves[h][0]) for h in range(_NSPLIT)],
        axis=0,
    )

    agent_out = jnp.concatenate(
        [_mm(halves[h][0], agent_w, agent_b[None, :])
         for h in range(_NSPLIT)],
        axis=0,
    )
    my_out = _myfc(my_emb, my_w, my_b[None, :])
    return agent_out, my_out


# speedup vs baseline: 1.1821x; 1.0394x over previous
import functools

import jax
import jax.numpy as jnp
from jax import lax
from jax.experimental import pallas as pl
from jax.experimental.pallas import tpu as pltpu
from jax.experimental.pallas import tpu_sc as plsc

_B = 1024
_NSPLIT = 1
_BH = _B // _NSPLIT
_A = 100
_AP = 104
_ATTRS = 23
_EMB = 32
_FAN = _ATTRS * _EMB
_RPH = _BH * _AP
_HID = 512
_NC, _NS = 2, 16
_NW = _NC * _NS
_CB = 2
_NCHUNK = _BH // (_NW * _CB)
_SLOT = _AP * _ATTRS
_SLOTP = 2400

_mesh = functools.partial(plsc.VectorSubcoreMesh,
                          core_axis_name="c", subcore_axis_name="s")

_SC_PARAMS = pltpu.CompilerParams(
    use_tc_tiling_on_sc=False, needs_layout_passes=False
)


def _wid():
    return lax.axis_index("s") * _NC + lax.axis_index("c")


def _make_gather(h):
    @functools.partial(
        pl.kernel,
        mesh=_mesh(),
        compiler_params=_SC_PARAMS,
        out_type=(
            jax.ShapeDtypeStruct((_RPH, _FAN), jnp.float32),
            jax.ShapeDtypeStruct((_BH * 128,), jnp.int32),
        ),
        scratch_types=[
            pltpu.VMEM((_CB * _SLOTP,), jnp.int32),
            pltpu.VMEM((_CB * _SLOTP,), jnp.int32),
            pltpu.VMEM((_CB * _SLOTP,), jnp.int32),
            pltpu.VMEM((_CB * _SLOTP,), jnp.int32),
            pltpu.VMEM((_CB, _A, _ATTRS), jnp.int32),
            pltpu.VMEM((_CB * _SLOTP,), jnp.int32),
            pltpu.VMEM((_SLOT, _EMB), jnp.float32),
            pltpu.SemaphoreType.DMA,
            pltpu.SemaphoreType.DMA,
        ],
        name=f"sc_gather_{h}",
    )
    def _gather(ag_hbm, pq_hbm, pr_hbm, pj_hbm, off_hbm, table_hbm,
                out_hbm, ids_hbm,
                pq_v, pr_v, pj_v, off_v, ag_v, idx_v, rows_v, sem_g, sem_o):
        pltpu.sync_copy(pq_hbm, pq_v)
        pltpu.sync_copy(pr_hbm, pr_v)
        pltpu.sync_copy(pj_hbm, pj_v)
        pltpu.sync_copy(off_hbm, off_v)
        w = _wid()

        def chunk_body(ci, carry):
            gc = w * _NCHUNK + ci
            pltpu.sync_copy(
                ag_hbm.at[pl.ds((h * _BH // _CB + gc) * _CB, _CB)], ag_v
            )

            def reorder(s, c2):
                s16 = s * 16
                sl = pl.ds(s16, 16)
                vals = plsc.load_gather(
                    ag_v, [pq_v[sl], pr_v[sl], pj_v[sl]]
                )
                vals = jnp.minimum(jnp.maximum(vals, 0), 255)
                idx_v[sl] = vals + off_v[sl]
                return c2

            lax.fori_loop(0, (_CB * _SLOTP) // 16, reorder, 0)

            def batch_body(q, c3):
                gb = gc * _CB + q
                pltpu.sync_copy(
                    idx_v.at[pl.ds(q * _SLOTP, _AP)],
                    ids_hbm.at[pl.ds(gb * 128, _AP)],
                )
                gathers = [
                    pltpu.async_copy(
                        table_hbm.at[
                            idx_v.at[pl.ds(q * _SLOTP + j * _AP, _AP)]
                        ],
                        rows_v.at[pl.ds(j * _AP, _AP)],
                        sem_g,
                    )
                    for j in range(_ATTRS)
                ]
                for c in gathers:
                    c.wait()
                outs = [
                    pltpu.async_copy(
                        rows_v.at[pl.ds(j * _AP, _AP)],
                        out_hbm.at[
                            pl.ds(gb * _AP, _AP), pl.ds(j * _EMB, _EMB)
                        ],
                        sem_o,
                    )
                    for j in range(_ATTRS)
                ]
                for c in outs:
                    c.wait()
                return c3

            lax.fori_loop(0, _CB, batch_body, 0)
            return carry

        lax.fori_loop(0, _NCHUNK, chunk_body, 0)

    return _gather


_sc_gather = [_make_gather(h) for h in range(_NSPLIT)]


_BWH = _BH // _NW


def _make_my_gather(h):
    @functools.partial(
        pl.kernel,
        mesh=_mesh(),
        compiler_params=_SC_PARAMS,
        out_type=jax.ShapeDtypeStruct((_BH, _FAN), jnp.float32),
        scratch_types=[
            pltpu.VMEM((_BWH, 8), jnp.int32),
            pltpu.VMEM((_BWH,), jnp.int32),
            pltpu.VMEM((_BWH, _FAN), jnp.float32),
            pltpu.SemaphoreType.DMA,
        ],
        name=f"sc_my_gather_{h}",
    )
    def _my_gather(g_hbm, emb_hbm, out_hbm, g8_v, g_v, rows_v, sem):
        base = _wid() * _BWH
        pltpu.sync_copy(
            g_hbm.at[pl.ds(h * _BH + base, _BWH), pl.ds(0, 8)], g8_v
        )
        for t in range(_BWH // 16):
            pos = lax.iota(jnp.int32, 16) + t * 16
            zero = jnp.zeros((16,), jnp.int32)
            g_v[pl.ds(t * 16, 16)] = (
                plsc.load_gather(g8_v, [pos, zero]) - h * _RPH
            )
        pltpu.async_copy(emb_hbm.at[g_v], rows_v, sem).wait()
        pltpu.sync_copy(rows_v, out_hbm.at[pl.ds(base, _BWH)])

    return _my_gather


_sc_my_gather = [_make_my_gather(h) for h in range(_NSPLIT)]


def _rowsel_body(ids_ref, my_ref, g_ref):
    ids = ids_ref[...]
    my = my_ref[...][:, :1]
    lane = lax.broadcasted_iota(jnp.int32, ids.shape, 1)
    match = (ids == my) & (ids != 0) & (lane < _A)
    cand = jnp.where(match, lane, 16384)
    row = jnp.min(cand, axis=1, keepdims=True)
    row = jnp.where(row >= 16384, 0, row)
    bidx = lax.broadcasted_iota(jnp.int32, row.shape, 0)
    g_ref[...] = jnp.broadcast_to(bidx * _AP + row, ids.shape)


def _rowsel(ids2, my8):
    return pl.pallas_call(
        _rowsel_body,
        out_shape=jax.ShapeDtypeStruct((_B, 128), jnp.int32),
    )(ids2, my8)


_BB = 16
_XB = _BB * _AP
_DN = (((1,), (1,)), ((), ()))


def _mm_body(x_ref, w_ref, b_ref, o_ref):
    res = (
        lax.dot_general(
            x_ref[...], w_ref[...], _DN,
            preferred_element_type=jnp.float32,
        )
        + b_ref[...]
    )
    for k in range(_BB):
        o_ref[k] = res[k * _AP : k * _AP + _A]


def _mm(emb, w, bias):
    return pl.pallas_call(
        _mm_body,
        grid=(_BH // _BB,),
        in_specs=[
            pl.BlockSpec((_XB, _FAN), lambda i: (i, 0)),
            pl.BlockSpec((_HID, _FAN), lambda i: (0, 0)),
            pl.BlockSpec((1, _HID), lambda i: (0, 0)),
        ],
        out_specs=pl.BlockSpec((_BB, _A, _HID), lambda i: (i, 0, 0)),
        out_shape=jax.ShapeDtypeStruct((_BH, _A, _HID), jnp.float32),
        compiler_params=pltpu.CompilerParams(
            dimension_semantics=("arbitrary",)
        ),
    )(emb, w, bias)


def _myfc_body(x_ref, w_ref, b_ref, o_ref):
    o_ref[...] = jnp.maximum(
        lax.dot_general(
            x_ref[...], w_ref[...], _DN,
            preferred_element_type=jnp.float32,
        )
        + b_ref[...],
        0.0,
    )


def _myfc(x, w, bias):
    return pl.pallas_call(
        _myfc_body,
        out_shape=jax.ShapeDtypeStruct((_B, _HID), jnp.float32),
    )(x, w, bias)


def _make_maps():
    s = jnp.arange(_CB * _SLOTP, dtype=jnp.int32)
    q = s // _SLOTP
    t = s % _SLOTP
    j = t // _AP
    r = t % _AP
    valid = (j < _ATTRS) & (r < _A)
    pq = jnp.where(valid, q, 0)
    pr = jnp.where(valid, r, 0)
    pj = jnp.where(valid, j, 0)
    off = jnp.where(j < _ATTRS, j * 256, 0)
    return pq, pr, pj, off


def kernel(agents, my_id, emb_table, agent_w, agent_b, my_w, my_b):
    pq, pr, pj, off = _make_maps()
    halves = [
        _sc_gather[h](agents, pq, pr, pj, off, emb_table)
        for h in range(_NSPLIT)
    ]

    my8 = jnp.broadcast_to(my_id[:, None], (_B, 8))
    ids2 = jnp.concatenate(
        [ids.reshape(_BH, 128) for _, ids in halves], axis=0
    )
    g2 = _rowsel(ids2, my8)

    my_emb = jnp.concatenate(
        [_sc_my_gather[h](g2, halves[h][0]) for h in range(_NSPLIT)],
        axis=0,
    )

    agent_out = jnp.concatenate(
        [_mm(halves[h][0], agent_w, agent_b[None, :])
         for h in range(_NSPLIT)],
        axis=0,
    )
    my_out = _myfc(my_emb, my_w, my_b[None, :])
    return agent_out, my_out


# speedup vs baseline: 1.1883x; 1.0052x over previous
import functools

import jax
import jax.numpy as jnp
from jax import lax
from jax.experimental import pallas as pl
from jax.experimental.pallas import tpu as pltpu
from jax.experimental.pallas import tpu_sc as plsc

_B = 1024
_NSPLIT = 1
_BH = _B // _NSPLIT
_A = 100
_AP = 104
_ATTRS = 23
_EMB = 32
_FAN = _ATTRS * _EMB
_RPH = _BH * _AP
_HID = 512
_NC, _NS = 2, 16
_NW = _NC * _NS
_CB = 2
_NCHUNK = _BH // (_NW * _CB)
_SLOT = _AP * _ATTRS
_SLOTP = 2400

_mesh = functools.partial(plsc.VectorSubcoreMesh,
                          core_axis_name="c", subcore_axis_name="s")

_SC_PARAMS = pltpu.CompilerParams(
    use_tc_tiling_on_sc=False, needs_layout_passes=False
)


def _wid():
    return lax.axis_index("s") * _NC + lax.axis_index("c")


def _make_gather(h):
    @functools.partial(
        pl.kernel,
        mesh=_mesh(),
        compiler_params=_SC_PARAMS,
        out_type=(
            jax.ShapeDtypeStruct((_RPH, _FAN), jnp.float32),
            jax.ShapeDtypeStruct((_BH * 128,), jnp.int32),
        ),
        scratch_types=[
            pltpu.VMEM((_CB * _SLOTP,), jnp.int32),
            pltpu.VMEM((_CB * _SLOTP,), jnp.int32),
            pltpu.VMEM((_CB * _SLOTP,), jnp.int32),
            pltpu.VMEM((_CB * _SLOTP,), jnp.int32),
            pltpu.VMEM((_CB, _A, _ATTRS), jnp.int32),
            pltpu.VMEM((_CB * _SLOTP,), jnp.int32),
            pltpu.VMEM((_SLOT, _EMB), jnp.float32),
            pltpu.SemaphoreType.DMA,
            pltpu.SemaphoreType.DMA,
        ],
        name=f"sc_gather_{h}",
    )
    def _gather(ag_hbm, pq_hbm, pr_hbm, pj_hbm, off_hbm, table_hbm,
                out_hbm, ids_hbm,
                pq_v, pr_v, pj_v, off_v, ag_v, idx_v, rows_v, sem_g, sem_o):
        pltpu.sync_copy(pq_hbm, pq_v)
        pltpu.sync_copy(pr_hbm, pr_v)
        pltpu.sync_copy(pj_hbm, pj_v)
        pltpu.sync_copy(off_hbm, off_v)
        w = _wid()

        def chunk_body(ci, carry):
            gc = w * _NCHUNK + ci
            pltpu.sync_copy(
                ag_hbm.at[pl.ds((h * _BH // _CB + gc) * _CB, _CB)], ag_v
            )

            def reorder(s, c2):
                s16 = s * 16
                sl = pl.ds(s16, 16)
                vals = plsc.load_gather(
                    ag_v, [pq_v[sl], pr_v[sl], pj_v[sl]]
                )
                vals = jnp.minimum(jnp.maximum(vals, 0), 255)
                idx_v[sl] = vals + off_v[sl]
                return c2

            lax.fori_loop(0, (_CB * _SLOTP) // 16, reorder, 0)

            def batch_body(q, c3):
                gb = gc * _CB + q
                pltpu.sync_copy(
                    idx_v.at[pl.ds(q * _SLOTP, _AP)],
                    ids_hbm.at[pl.ds(gb * 128, _AP)],
                )
                gathers = [
                    pltpu.async_copy(
                        table_hbm.at[
                            idx_v.at[pl.ds(q * _SLOTP + j * _AP, _AP)]
                        ],
                        rows_v.at[pl.ds(j * _AP, _AP)],
                        sem_g,
                    )
                    for j in range(_ATTRS)
                ]
                for c in gathers:
                    c.wait()
                outs = [
                    pltpu.async_copy(
                        rows_v.at[pl.ds(j * _AP, _AP)],
                        out_hbm.at[
                            pl.ds(gb * _AP, _AP), pl.ds(j * _EMB, _EMB)
                        ],
                        sem_o,
                    )
                    for j in range(_ATTRS)
                ]
                for c in outs:
                    c.wait()
                return c3

            lax.fori_loop(0, _CB, batch_body, 0)
            return carry

        lax.fori_loop(0, _NCHUNK, chunk_body, 0)

    return _gather


_sc_gather = [_make_gather(h) for h in range(_NSPLIT)]


_BWH = _BH // _NW


def _make_my_gather(h):
    @functools.partial(
        pl.kernel,
        mesh=_mesh(),
        compiler_params=_SC_PARAMS,
        out_type=jax.ShapeDtypeStruct((_BH, _FAN), jnp.float32),
        scratch_types=[
            pltpu.VMEM((_BWH, 8), jnp.int32),
            pltpu.VMEM((_BWH,), jnp.int32),
            pltpu.VMEM((_BWH, _FAN), jnp.float32),
            pltpu.SemaphoreType.DMA,
        ],
        name=f"sc_my_gather_{h}",
    )
    def _my_gather(g_hbm, emb_hbm, out_hbm, g8_v, g_v, rows_v, sem):
        base = _wid() * _BWH
        pltpu.sync_copy(
            g_hbm.at[pl.ds(h * _BH + base, _BWH), pl.ds(0, 8)], g8_v
        )
        for t in range(_BWH // 16):
            pos = lax.iota(jnp.int32, 16) + t * 16
            zero = jnp.zeros((16,), jnp.int32)
            g_v[pl.ds(t * 16, 16)] = (
                plsc.load_gather(g8_v, [pos, zero]) - h * _RPH
            )
        pltpu.async_copy(emb_hbm.at[g_v], rows_v, sem).wait()
        pltpu.sync_copy(rows_v, out_hbm.at[pl.ds(base, _BWH)])

    return _my_gather


_sc_my_gather = [_make_my_gather(h) for h in range(_NSPLIT)]


def _rowsel_body(ids_ref, my_ref, g_ref):
    ids = ids_ref[...]
    my = my_ref[...][:, :1]
    lane = lax.broadcasted_iota(jnp.int32, ids.shape, 1)
    match = (ids == my) & (ids != 0) & (lane < _A)
    cand = jnp.where(match, lane, 16384)
    row = jnp.min(cand, axis=1, keepdims=True)
    row = jnp.where(row >= 16384, 0, row)
    bidx = lax.broadcasted_iota(jnp.int32, row.shape, 0)
    g_ref[...] = jnp.broadcast_to(bidx * _AP + row, ids.shape)


def _rowsel(ids2, my8):
    return pl.pallas_call(
        _rowsel_body,
        out_shape=jax.ShapeDtypeStruct((_B, 128), jnp.int32),
    )(ids2, my8)


_BB = 32
_XB = _BB * _AP
_DN = (((1,), (1,)), ((), ()))


def _mm_body(x_ref, w_ref, b_ref, o_ref):
    res = (
        lax.dot_general(
            x_ref[...], w_ref[...], _DN,
            preferred_element_type=jnp.float32,
        )
        + b_ref[...]
    )
    for k in range(_BB):
        o_ref[k] = res[k * _AP : k * _AP + _A]


def _mm(emb, w, bias):
    return pl.pallas_call(
        _mm_body,
        grid=(_BH // _BB,),
        in_specs=[
            pl.BlockSpec((_XB, _FAN), lambda i: (i, 0)),
            pl.BlockSpec((_HID, _FAN), lambda i: (0, 0)),
            pl.BlockSpec((1, _HID), lambda i: (0, 0)),
        ],
        out_specs=pl.BlockSpec((_BB, _A, _HID), lambda i: (i, 0, 0)),
        out_shape=jax.ShapeDtypeStruct((_BH, _A, _HID), jnp.float32),
        compiler_params=pltpu.CompilerParams(
            dimension_semantics=("arbitrary",)
        ),
    )(emb, w, bias)


def _myfc_body(x_ref, w_ref, b_ref, o_ref):
    o_ref[...] = jnp.maximum(
        lax.dot_general(
            x_ref[...], w_ref[...], _DN,
            preferred_element_type=jnp.float32,
        )
        + b_ref[...],
        0.0,
    )


def _myfc(x, w, bias):
    return pl.pallas_call(
        _myfc_body,
        out_shape=jax.ShapeDtypeStruct((_B, _HID), jnp.float32),
    )(x, w, bias)


def _make_maps():
    s = jnp.arange(_CB * _SLOTP, dtype=jnp.int32)
    q = s // _SLOTP
    t = s % _SLOTP
    j = t // _AP
    r = t % _AP
    valid = (j < _ATTRS) & (r < _A)
    pq = jnp.where(valid, q, 0)
    pr = jnp.where(valid, r, 0)
    pj = jnp.where(valid, j, 0)
    off = jnp.where(j < _ATTRS, j * 256, 0)
    return pq, pr, pj, off


def kernel(agents, my_id, emb_table, agent_w, agent_b, my_w, my_b):
    pq, pr, pj, off = _make_maps()
    halves = [
        _sc_gather[h](agents, pq, pr, pj, off, emb_table)
        for h in range(_NSPLIT)
    ]

    my8 = jnp.broadcast_to(my_id[:, None], (_B, 8))
    ids2 = jnp.concatenate(
        [ids.reshape(_BH, 128) for _, ids in halves], axis=0
    )
    g2 = _rowsel(ids2, my8)

    my_emb = jnp.concatenate(
        [_sc_my_gather[h](g2, halves[h][0]) for h in range(_NSPLIT)],
        axis=0,
    )

    agent_out = jnp.concatenate(
        [_mm(halves[h][0], agent_w, agent_b[None, :])
         for h in range(_NSPLIT)],
        axis=0,
    )
    my_out = _myfc(my_emb, my_w, my_b[None, :])
    return agent_out, my_out
